# Initial kernel scaffold; baseline (speedup 1.0000x reference)
import jax
import jax.numpy as jnp
from jax.experimental import pallas as pl


def kernel(x, edge_index, edge_weight, batch, emb_table, W1_rel, b1_rel, W1_root, Wmu_rel, bmu_rel, Wmu_root, Wstd_rel, bstd_rel, Wstd_root, Wc1, bc1, Wc2, bc2, log_std, eps):
    raise NotImplementedError("write your pallas kernel here")


# speedup vs baseline: 1.0404x; 1.0404x over previous
import functools

import jax
import jax.numpy as jnp
from jax import lax
from jax.experimental import pallas as pl
from jax.experimental.pallas import tpu as pltpu

N = 10000
E = 320000
HIDDEN = 128
EMB1 = 128
EMB2 = 64
L1 = 64
G = 64


def _tc1_body(e_raw_ref, w1root_ref, b1_ref, e_ref, eroot_ref):
    e_raw = e_raw_ref[:]
    nrm2 = jnp.sum(e_raw * e_raw, axis=1, keepdims=True)
    scale = jnp.where(nrm2 > 1.0, lax.rsqrt(nrm2), 1.0)
    e = e_raw * scale
    e_ref[:] = e
    eroot_ref[:] = (
        jax.lax.dot_general(e, w1root_ref[:], (((1,), (1,)), ((), ())),
                            preferred_element_type=jnp.float32)
        + b1_ref[:][None, :]
    )


def _tc1(e_raw, W1_root, b1_rel):
    return pl.pallas_call(
        _tc1_body,
        out_shape=(
            jax.ShapeDtypeStruct((N, HIDDEN), jnp.float32),
            jax.ShapeDtypeStruct((N, EMB1), jnp.float32),
        ),
    )(e_raw, W1_root, b1_rel)


def _tc3_body(agg_ref, w1rel_ref, eroot_ref, h_ref):
    h = (
        jax.lax.dot_general(agg_ref[:], w1rel_ref[:], (((1,), (1,)), ((), ())),
                            preferred_element_type=jnp.float32)
        + eroot_ref[:]
    )
    h_ref[:] = jnp.maximum(h, 0.0)


def _tc3(agg1, W1_rel, eroot):
    return pl.pallas_call(
        _tc3_body,
        out_shape=jax.ShapeDtypeStruct((N, EMB1), jnp.float32),
    )(agg1, W1_rel, eroot)


def _tc5_body(agg2_ref, h_ref, wmu_rel_ref, bmu_ref, wmu_root_ref,
              wstd_rel_ref, bstd_ref, wstd_root_ref, eps_ref, batch_ref,
              wc1_ref, bc1_ref, wc2_ref, bc2_ref, logstd_ref,
              z_ref, zmu_ref, zstd_ref, zn_ref, y_ref, wstd_out_ref):
    agg2 = agg2_ref[:]
    h = h_ref[:]

    def mm_t(a, w):
        return jax.lax.dot_general(a, w, (((1,), (1,)), ((), ())),
                                   preferred_element_type=jnp.float32)

    z_mu = jnp.tanh(mm_t(agg2, wmu_rel_ref[:]) + bmu_ref[:][None, :]
                    + mm_t(h, wmu_root_ref[:]))
    z_ls = jnp.tanh(mm_t(agg2, wstd_rel_ref[:]) + bstd_ref[:][None, :]
                    + mm_t(h, wstd_root_ref[:]))
    z_std = jnp.exp(z_ls)
    z = z_mu + z_std * eps_ref[:]
    zmu_ref[:] = z_mu
    zstd_ref[:] = z_std
    z_ref[:] = z
    zn2 = jnp.sum(z * z, axis=1, keepdims=True)
    rinv = 1.0 / jnp.maximum(jnp.sqrt(zn2), 1e-8)
    zn_ref[:] = z * rinv
    seg = lax.broadcasted_iota(jnp.int32, (G, N), 0)
    mask = (batch_ref[:][None, :] == seg).astype(jnp.float32)
    cnt = jnp.sum(mask, axis=1, keepdims=True)
    pooled = jax.lax.dot_general(mask, z_mu, (((1,), (0,)), ((), ())),
                                 preferred_element_type=jnp.float32)
    pooled = pooled / jnp.maximum(cnt, 1.0)
    y = jnp.maximum(mm_t(pooled, wc1_ref[:]) + bc1_ref[:][None, :], 0.0)
    y = mm_t(y, wc2_ref[:]) + bc2_ref[:][None, :]
    y = y - jnp.max(y, axis=1, keepdims=True)
    ey = jnp.exp(y)
    y_ref[:] = ey / jnp.sum(ey, axis=1, keepdims=True)
    wstd_out_ref[:] = jnp.exp(logstd_ref[:])


def _tc5(agg2, h, Wmu_rel, bmu_rel, Wmu_root, Wstd_rel, bstd_rel, Wstd_root,
         eps, batch, Wc1, bc1, Wc2, bc2, log_std):
    return pl.pallas_call(
        _tc5_body,
        out_shape=(
            jax.ShapeDtypeStruct((N, EMB2), jnp.float32),
            jax.ShapeDtypeStruct((N, EMB2), jnp.float32),
            jax.ShapeDtypeStruct((N, EMB2), jnp.float32),
            jax.ShapeDtypeStruct((N, EMB2), jnp.float32),
            jax.ShapeDtypeStruct((G, 2), jnp.float32),
            jax.ShapeDtypeStruct((1,), jnp.float32),
        ),
    )(agg2, h, Wmu_rel, bmu_rel, Wmu_root, Wstd_rel, bstd_rel, Wstd_root,
      eps, batch, Wc1, bc1, Wc2, bc2, log_std)


def _gather_rows(table, idx):
    return table[idx]


def _segsum(xrows, ew, dst):
    return jax.ops.segment_sum(xrows * ew[:, None], dst, num_segments=N)


def _edge_dot(zn, src, dst):
    return jnp.sum(zn[src] * zn[dst], axis=1)


def kernel(x, edge_index, edge_weight, batch, emb_table, W1_rel, b1_rel,
           W1_root, Wmu_rel, bmu_rel, Wmu_root, Wstd_rel, bstd_rel, Wstd_root,
           Wc1, bc1, Wc2, bc2, log_std, eps):
    src = edge_index[0]
    dst = edge_index[1]

    e_raw = _gather_rows(emb_table, x)
    e, eroot = _tc1(e_raw, W1_root, b1_rel)

    agg1 = _segsum(_gather_rows(e, src), edge_weight, dst)
    h = _tc3(agg1, W1_rel, eroot)

    agg2 = _segsum(_gather_rows(h, src), edge_weight, dst)
    z, z_mu, z_std, zn, y, w_std = _tc5(
        agg2, h, Wmu_rel, bmu_rel, Wmu_root, Wstd_rel, bstd_rel, Wstd_root,
        eps, batch, Wc1, bc1, Wc2, bc2, log_std)

    w_mu = _edge_dot(zn, src, dst)
    return (y, w_mu, w_std, z, z_mu, z_std)


# speedup vs baseline: 1.9855x; 1.9084x over previous
import functools

import jax
import jax.numpy as jnp
from jax import lax
from jax.experimental import pallas as pl
from jax.experimental.pallas import tpu as pltpu
from jax.experimental.pallas import tpu_sc as plsc

_NC = 2
_NS = 16
_NW = _NC * _NS
_LANES = 16

N = 10000
E = 320000
HIDDEN = 128
EMB1 = 128
EMB2 = 64
L1 = 64
G = 64


def _tc1_body(e_raw_ref, w1root_ref, b1_ref, e_ref, eroot_ref):
    e_raw = e_raw_ref[:]
    nrm2 = jnp.sum(e_raw * e_raw, axis=1, keepdims=True)
    scale = jnp.where(nrm2 > 1.0, lax.rsqrt(nrm2), 1.0)
    e = e_raw * scale
    e_ref[:] = e
    eroot_ref[:] = (
        jax.lax.dot_general(e, w1root_ref[:], (((1,), (1,)), ((), ())),
                            preferred_element_type=jnp.float32)
        + b1_ref[:][None, :]
    )


def _tc1(e_raw, W1_root, b1_rel):
    return pl.pallas_call(
        _tc1_body,
        out_shape=(
            jax.ShapeDtypeStruct((N, HIDDEN), jnp.float32),
            jax.ShapeDtypeStruct((N, EMB1), jnp.float32),
        ),
    )(e_raw, W1_root, b1_rel)


def _tc3_body(aggp_ref, w1rel_ref, eroot_ref, h_ref):
    agg = aggp_ref[0:N, :] + aggp_ref[_NROW:_NROW + N, :]
    h = (
        jax.lax.dot_general(agg, w1rel_ref[:], (((1,), (1,)), ((), ())),
                            preferred_element_type=jnp.float32)
        + eroot_ref[:]
    )
    h_ref[:] = jnp.maximum(h, 0.0)


def _tc3(agg1p, W1_rel, eroot):
    return pl.pallas_call(
        _tc3_body,
        out_shape=jax.ShapeDtypeStruct((N, EMB1), jnp.float32),
    )(agg1p, W1_rel, eroot)


def _tc5_body(agg2_ref, h_ref, wmu_rel_ref, bmu_ref, wmu_root_ref,
              wstd_rel_ref, bstd_ref, wstd_root_ref, eps_ref, batch_ref,
              wc1_ref, bc1_ref, wc2_ref, bc2_ref, logstd_ref,
              z_ref, zmu_ref, zstd_ref, zn_ref, y_ref, wstd_out_ref):
    agg2 = agg2_ref[0:N, :] + agg2_ref[_NROW:_NROW + N, :]
    h = h_ref[:]

    def mm_t(a, w):
        return jax.lax.dot_general(a, w, (((1,), (1,)), ((), ())),
                                   preferred_element_type=jnp.float32)

    z_mu = jnp.tanh(mm_t(agg2, wmu_rel_ref[:]) + bmu_ref[:][None, :]
                    + mm_t(h, wmu_root_ref[:]))
    z_ls = jnp.tanh(mm_t(agg2, wstd_rel_ref[:]) + bstd_ref[:][None, :]
                    + mm_t(h, wstd_root_ref[:]))
    z_std = jnp.exp(z_ls)
    z = z_mu + z_std * eps_ref[:]
    zmu_ref[:] = z_mu
    zstd_ref[:] = z_std
    z_ref[:] = z
    zn2 = jnp.sum(z * z, axis=1, keepdims=True)
    rinv = 1.0 / jnp.maximum(jnp.sqrt(zn2), 1e-8)
    zn_ref[:] = z * rinv
    seg = lax.broadcasted_iota(jnp.int32, (G, N), 0)
    mask = (batch_ref[:][None, :] == seg).astype(jnp.float32)
    cnt = jnp.sum(mask, axis=1, keepdims=True)
    pooled = jax.lax.dot_general(mask, z_mu, (((1,), (0,)), ((), ())),
                                 preferred_element_type=jnp.float32)
    pooled = pooled / jnp.maximum(cnt, 1.0)
    y = jnp.maximum(mm_t(pooled, wc1_ref[:]) + bc1_ref[:][None, :], 0.0)
    y = mm_t(y, wc2_ref[:]) + bc2_ref[:][None, :]
    y = y - jnp.max(y, axis=1, keepdims=True)
    ey = jnp.exp(y)
    y_ref[:] = ey / jnp.sum(ey, axis=1, keepdims=True)
    wstd_out_ref[:] = jnp.exp(logstd_ref[:])


def _tc5(agg2, h, Wmu_rel, bmu_rel, Wmu_root, Wstd_rel, bstd_rel, Wstd_root,
         eps, batch, Wc1, bc1, Wc2, bc2, log_std):
    return pl.pallas_call(
        _tc5_body,
        out_shape=(
            jax.ShapeDtypeStruct((N, EMB2), jnp.float32),
            jax.ShapeDtypeStruct((N, EMB2), jnp.float32),
            jax.ShapeDtypeStruct((N, EMB2), jnp.float32),
            jax.ShapeDtypeStruct((N, EMB2), jnp.float32),
            jax.ShapeDtypeStruct((G, 2), jnp.float32),
            jax.ShapeDtypeStruct((1,), jnp.float32),
        ),
    )(agg2, h, Wmu_rel, bmu_rel, Wmu_root, Wstd_rel, bstd_rel, Wstd_root,
      eps, batch, Wc1, bc1, Wc2, bc2, log_std)


def _chunk_sizes(total, cap=128):
    out = []
    while total > 0:
        c = min(cap, total)
        out.append(c)
        total -= c
    return out


@functools.partial(jax.jit, static_argnames=("n_rows", "n_cols"))
def _sc_gather(table, idx, n_rows, n_cols):
    bpw = n_rows // _NW
    mesh = plsc.VectorSubcoreMesh(core_axis_name="c", subcore_axis_name="s")

    @functools.partial(
        pl.kernel, mesh=mesh,
        out_type=jax.ShapeDtypeStruct((n_rows, n_cols), jnp.float32),
        scratch_types=[
            pltpu.VMEM((bpw,), jnp.int32),
            pltpu.VMEM((bpw, n_cols), jnp.float32),
            pltpu.SemaphoreType.DMA,
        ],
    )
    def k(table_hbm, idx_hbm, out_hbm, idx_v, rows_v, sem):
        wid = lax.axis_index("s") * _NC + lax.axis_index("c")
        base = wid * bpw
        pltpu.sync_copy(idx_hbm.at[pl.ds(base, bpw)], idx_v)
        copies = []
        off = 0
        for cs in _chunk_sizes(bpw):
            copies.append(pltpu.async_copy(
                table_hbm.at[idx_v.at[pl.ds(off, cs)]],
                rows_v.at[pl.ds(off, cs)], sem))
            off += cs
        for c in copies:
            c.wait()
        pltpu.sync_copy(rows_v, out_hbm.at[pl.ds(base, bpw)])

    return k(table, idx)


def _gather_rows(table, idx):
    return table[idx]


_C = 128
_NCH = 79
_EPW = _NCH * _C
_E_PAD = _NW * _EPW
_NROW = 10240
_RPS = _NROW // _NS


@functools.partial(jax.jit, static_argnames=("n_cols",))
def _sc_segsum(table, srcp, dstp, ewp, zeros_tbl, n_cols):
    mesh = plsc.VectorSubcoreMesh(core_axis_name="c", subcore_axis_name="s")
    cvecs = n_cols // _LANES

    @functools.partial(
        pl.kernel, mesh=mesh,
        out_type=jax.ShapeDtypeStruct((2 * _NROW, n_cols), jnp.float32),
        compiler_params=pltpu.CompilerParams(needs_layout_passes=False),
        scratch_types=[
            pltpu.VMEM((_NCH, _C), jnp.int32),
            pltpu.VMEM((_NCH, _C), jnp.int32),
            pltpu.VMEM((_EPW,), jnp.float32),
            pltpu.VMEM((_C, n_cols), jnp.float32),
            pltpu.VMEM_SHARED((_NROW, n_cols), jnp.float32),
            pltpu.SemaphoreType.DMA,
        ],
    )
    def k(x_hbm, src_hbm, dst_hbm, ew_hbm, z_hbm, out_hbm,
          src_v, dst_v, ew_v, msg_v, agg_sh, sem):
        cid = lax.axis_index("c")
        sid = lax.axis_index("s")
        wid = sid * _NC + cid
        pltpu.sync_copy(z_hbm.at[pl.ds(sid * _RPS, _RPS)],
                        agg_sh.at[pl.ds(sid * _RPS, _RPS)])
        pltpu.sync_copy(src_hbm.at[wid], src_v)
        pltpu.sync_copy(dst_hbm.at[wid], dst_v)
        pltpu.sync_copy(ew_hbm.at[wid], ew_v)
        plsc.subcore_barrier()

        def chunk(j, carry):
            pltpu.async_copy(x_hbm.at[src_v.at[j]], msg_v, sem).wait()
            jbase = jnp.full((_LANES,), j * _C, jnp.int32)

            def srow(i, c2):
                w = plsc.load_gather(ew_v, [jbase + i])
                for c in range(cvecs):
                    msg_v[i, pl.ds(c * _LANES, _LANES)] = (
                        msg_v[i, pl.ds(c * _LANES, _LANES)] * w)
                return c2

            lax.fori_loop(0, _C, srow, 0)
            pltpu.sync_copy(msg_v, agg_sh.at[dst_v.at[j]], add=True)
            return carry

        lax.fori_loop(0, _NCH, chunk, 0)
        plsc.subcore_barrier()
        pltpu.sync_copy(agg_sh.at[pl.ds(sid * _RPS, _RPS)],
                        out_hbm.at[pl.ds(cid * _NROW + sid * _RPS, _RPS)])

    return k(table, srcp, dstp, ewp, zeros_tbl)


def _segsum(xrows, ew, dst):
    return jax.ops.segment_sum(xrows * ew[:, None], dst, num_segments=N)


def _edge_dot(zn, src, dst):
    return jnp.sum(zn[src] * zn[dst], axis=1)


def kernel(x, edge_index, edge_weight, batch, emb_table, W1_rel, b1_rel,
           W1_root, Wmu_rel, bmu_rel, Wmu_root, Wstd_rel, bstd_rel, Wstd_root,
           Wc1, bc1, Wc2, bc2, log_std, eps):
    src = edge_index[0]
    dst = edge_index[1]

    n_pad = 10240
    x_pad = jnp.pad(x, (0, n_pad - N))
    e_raw = _sc_gather(emb_table, x_pad, n_pad, HIDDEN)[:N]
    e, eroot = _tc1(e_raw, W1_root, b1_rel)

    pe = _E_PAD - E
    srcp = jnp.pad(src, (0, pe)).reshape(_NW, _NCH, _C)
    dstp = jnp.pad(dst, (0, pe)).reshape(_NW, _NCH, _C)
    ewp = jnp.pad(edge_weight, (0, pe)).reshape(_NW, _EPW)
    zeros_tbl = jnp.zeros((_NROW, HIDDEN), jnp.float32)

    agg1p = _sc_segsum(e, srcp, dstp, ewp, zeros_tbl, HIDDEN)
    h = _tc3(agg1p, W1_rel, eroot)

    agg2p = _sc_segsum(h, srcp, dstp, ewp, zeros_tbl, EMB1)
    z, z_mu, z_std, zn, y, w_std = _tc5(
        agg2p, h, Wmu_rel, bmu_rel, Wmu_root, Wstd_rel, bstd_rel, Wstd_root,
        eps, batch, Wc1, bc1, Wc2, bc2, log_std)

    w_mu = _edge_dot(zn, src, dst)
    return (y, w_mu, w_std, z, z_mu, z_std)


# speedup vs baseline: 3.7958x; 1.9118x over previous
import functools

import jax
import jax.numpy as jnp
from jax import lax
from jax.experimental import pallas as pl
from jax.experimental.pallas import tpu as pltpu
from jax.experimental.pallas import tpu_sc as plsc

_NC = 2
_NS = 16
_NW = _NC * _NS
_LANES = 16

N = 10000
E = 320000
HIDDEN = 128
EMB1 = 128
EMB2 = 64
L1 = 64
G = 64


def _tc1_body(e_raw_ref, w1root_ref, b1_ref, e_ref, eroot_ref):
    e_raw = e_raw_ref[:]
    nrm2 = jnp.sum(e_raw * e_raw, axis=1, keepdims=True)
    scale = jnp.where(nrm2 > 1.0, lax.rsqrt(nrm2), 1.0)
    e = e_raw * scale
    e_ref[:] = e
    eroot_ref[:] = (
        jax.lax.dot_general(e, w1root_ref[:], (((1,), (1,)), ((), ())),
                            preferred_element_type=jnp.float32)
        + b1_ref[:][None, :]
    )


def _tc1(e_raw, W1_root, b1_rel):
    return pl.pallas_call(
        _tc1_body,
        out_shape=(
            jax.ShapeDtypeStruct((N, HIDDEN), jnp.float32),
            jax.ShapeDtypeStruct((N, EMB1), jnp.float32),
        ),
    )(e_raw, W1_root, b1_rel)


def _tc3_body(aggp_ref, w1rel_ref, eroot_ref, h_ref):
    agg = aggp_ref[0:N, :] + aggp_ref[_NROW:_NROW + N, :]
    h = (
        jax.lax.dot_general(agg, w1rel_ref[:], (((1,), (1,)), ((), ())),
                            preferred_element_type=jnp.float32)
        + eroot_ref[:]
    )
    h_ref[:] = jnp.maximum(h, 0.0)


def _tc3(agg1p, W1_rel, eroot):
    return pl.pallas_call(
        _tc3_body,
        out_shape=jax.ShapeDtypeStruct((N, EMB1), jnp.float32),
    )(agg1p, W1_rel, eroot)


def _tc5_body(agg2_ref, h_ref, wmu_rel_ref, bmu_ref, wmu_root_ref,
              wstd_rel_ref, bstd_ref, wstd_root_ref, eps_ref, batch_ref,
              wc1_ref, bc1_ref, wc2_ref, bc2_ref, logstd_ref,
              z_ref, zmu_ref, zstd_ref, zn_ref, y_ref, wstd_out_ref):
    agg2 = agg2_ref[0:N, :] + agg2_ref[_NROW:_NROW + N, :]
    h = h_ref[:]

    def mm_t(a, w):
        return jax.lax.dot_general(a, w, (((1,), (1,)), ((), ())),
                                   preferred_element_type=jnp.float32)

    z_mu = jnp.tanh(mm_t(agg2, wmu_rel_ref[:]) + bmu_ref[:][None, :]
                    + mm_t(h, wmu_root_ref[:]))
    z_ls = jnp.tanh(mm_t(agg2, wstd_rel_ref[:]) + bstd_ref[:][None, :]
                    + mm_t(h, wstd_root_ref[:]))
    z_std = jnp.exp(z_ls)
    z = z_mu + z_std * eps_ref[:]
    zmu_ref[:] = z_mu
    zstd_ref[:] = z_std
    z_ref[:] = z
    zn2 = jnp.sum(z * z, axis=1, keepdims=True)
    rinv = 1.0 / jnp.maximum(jnp.sqrt(zn2), 1e-8)
    zn_ref[:] = z * rinv
    seg = lax.broadcasted_iota(jnp.int32, (G, N), 0)
    mask = (batch_ref[:][None, :] == seg).astype(jnp.float32)
    cnt = jnp.sum(mask, axis=1, keepdims=True)
    pooled = jax.lax.dot_general(mask, z_mu, (((1,), (0,)), ((), ())),
                                 preferred_element_type=jnp.float32)
    pooled = pooled / jnp.maximum(cnt, 1.0)
    y = jnp.maximum(mm_t(pooled, wc1_ref[:]) + bc1_ref[:][None, :], 0.0)
    y = mm_t(y, wc2_ref[:]) + bc2_ref[:][None, :]
    y = y - jnp.max(y, axis=1, keepdims=True)
    ey = jnp.exp(y)
    y_ref[:] = ey / jnp.sum(ey, axis=1, keepdims=True)
    wstd_out_ref[:] = jnp.exp(logstd_ref[:])


def _tc5(agg2, h, Wmu_rel, bmu_rel, Wmu_root, Wstd_rel, bstd_rel, Wstd_root,
         eps, batch, Wc1, bc1, Wc2, bc2, log_std):
    return pl.pallas_call(
        _tc5_body,
        out_shape=(
            jax.ShapeDtypeStruct((N, EMB2), jnp.float32),
            jax.ShapeDtypeStruct((N, EMB2), jnp.float32),
            jax.ShapeDtypeStruct((N, EMB2), jnp.float32),
            jax.ShapeDtypeStruct((N, EMB2), jnp.float32),
            jax.ShapeDtypeStruct((G, 2), jnp.float32),
            jax.ShapeDtypeStruct((1,), jnp.float32),
        ),
    )(agg2, h, Wmu_rel, bmu_rel, Wmu_root, Wstd_rel, bstd_rel, Wstd_root,
      eps, batch, Wc1, bc1, Wc2, bc2, log_std)


def _chunk_sizes(total, cap=128):
    out = []
    while total > 0:
        c = min(cap, total)
        out.append(c)
        total -= c
    return out


@functools.partial(jax.jit, static_argnames=("n_rows", "n_cols"))
def _sc_gather(table, idx, n_rows, n_cols):
    bpw = n_rows // _NW
    mesh = plsc.VectorSubcoreMesh(core_axis_name="c", subcore_axis_name="s")

    @functools.partial(
        pl.kernel, mesh=mesh,
        out_type=jax.ShapeDtypeStruct((n_rows, n_cols), jnp.float32),
        scratch_types=[
            pltpu.VMEM((bpw,), jnp.int32),
            pltpu.VMEM((bpw, n_cols), jnp.float32),
            pltpu.SemaphoreType.DMA,
        ],
    )
    def k(table_hbm, idx_hbm, out_hbm, idx_v, rows_v, sem):
        wid = lax.axis_index("s") * _NC + lax.axis_index("c")
        base = wid * bpw
        pltpu.sync_copy(idx_hbm.at[pl.ds(base, bpw)], idx_v)
        copies = []
        off = 0
        for cs in _chunk_sizes(bpw):
            copies.append(pltpu.async_copy(
                table_hbm.at[idx_v.at[pl.ds(off, cs)]],
                rows_v.at[pl.ds(off, cs)], sem))
            off += cs
        for c in copies:
            c.wait()
        pltpu.sync_copy(rows_v, out_hbm.at[pl.ds(base, bpw)])

    return k(table, idx)


def _gather_rows(table, idx):
    return table[idx]


_C = 128
_NCH = 79
_EPW = _NCH * _C
_E_PAD = _NW * _EPW
_NROW = 10240
_RPS = _NROW // _NS


@functools.partial(jax.jit, static_argnames=("n_cols",))
def _sc_segsum(table, srcp, dstp, ewp, zeros_tbl, n_cols):
    mesh = plsc.VectorSubcoreMesh(core_axis_name="c", subcore_axis_name="s")
    cvecs = n_cols // _LANES

    @functools.partial(
        pl.kernel, mesh=mesh,
        out_type=jax.ShapeDtypeStruct((2 * _NROW, n_cols), jnp.float32),
        compiler_params=pltpu.CompilerParams(needs_layout_passes=False),
        scratch_types=[
            pltpu.VMEM((_NCH, _C), jnp.int32),
            pltpu.VMEM((_NCH, _C), jnp.int32),
            pltpu.VMEM((_EPW,), jnp.float32),
            pltpu.VMEM((_C, n_cols), jnp.float32),
            pltpu.VMEM_SHARED((_NROW, n_cols), jnp.float32),
            pltpu.SemaphoreType.DMA,
        ],
    )
    def k(x_hbm, src_hbm, dst_hbm, ew_hbm, z_hbm, out_hbm,
          src_v, dst_v, ew_v, msg_v, agg_sh, sem):
        cid = lax.axis_index("c")
        sid = lax.axis_index("s")
        wid = sid * _NC + cid
        pltpu.sync_copy(z_hbm.at[pl.ds(sid * _RPS, _RPS)],
                        agg_sh.at[pl.ds(sid * _RPS, _RPS)])
        pltpu.sync_copy(src_hbm.at[wid], src_v)
        pltpu.sync_copy(dst_hbm.at[wid], dst_v)
        pltpu.sync_copy(ew_hbm.at[wid], ew_v)
        plsc.subcore_barrier()

        def chunk(j, carry):
            pltpu.async_copy(x_hbm.at[src_v.at[j]], msg_v, sem).wait()
            jbase = jnp.full((_LANES,), j * _C, jnp.int32)

            def srow(i, c2):
                w = plsc.load_gather(ew_v, [jbase + i])
                for c in range(cvecs):
                    msg_v[i, pl.ds(c * _LANES, _LANES)] = (
                        msg_v[i, pl.ds(c * _LANES, _LANES)] * w)
                return c2

            lax.fori_loop(0, _C, srow, 0)
            pltpu.sync_copy(msg_v, agg_sh.at[dst_v.at[j]], add=True)
            return carry

        lax.fori_loop(0, _NCH, chunk, 0)
        plsc.subcore_barrier()
        pltpu.sync_copy(agg_sh.at[pl.ds(sid * _RPS, _RPS)],
                        out_hbm.at[pl.ds(cid * _NROW + sid * _RPS, _RPS)])

    return k(table, srcp, dstp, ewp, zeros_tbl)


def _segsum(xrows, ew, dst):
    return jax.ops.segment_sum(xrows * ew[:, None], dst, num_segments=N)


@jax.jit
def _sc_edge_dot(zn, srcp, dstp):
    mesh = plsc.VectorSubcoreMesh(core_axis_name="c", subcore_axis_name="s")
    cvecs = EMB2 // _LANES

    @functools.partial(
        pl.kernel, mesh=mesh,
        out_type=jax.ShapeDtypeStruct((_E_PAD,), jnp.float32),
        compiler_params=pltpu.CompilerParams(needs_layout_passes=False,
                                             use_tc_tiling_on_sc=False),
        scratch_types=[
            pltpu.VMEM((_NCH, _C), jnp.int32),
            pltpu.VMEM((_NCH, _C), jnp.int32),
            pltpu.VMEM((_C, EMB2), jnp.float32),
            pltpu.VMEM((_C, EMB2), jnp.float32),
            pltpu.VMEM((_EPW,), jnp.float32),
            pltpu.SemaphoreType.DMA,
        ],
    )
    def k(zn_hbm, src_hbm, dst_hbm, out_hbm, src_v, dst_v, a_v, b_v, o_v, sem):
        cid = lax.axis_index("c")
        sid = lax.axis_index("s")
        wid = sid * _NC + cid
        pltpu.sync_copy(src_hbm.at[wid], src_v)
        pltpu.sync_copy(dst_hbm.at[wid], dst_v)
        lane = lax.broadcasted_iota(jnp.int32, (_LANES,), 0)
        last = lane == (_LANES - 1)

        def chunk(j, carry):
            ca = pltpu.async_copy(zn_hbm.at[src_v.at[j]], a_v, sem)
            cb = pltpu.async_copy(zn_hbm.at[dst_v.at[j]], b_v, sem)
            ca.wait()
            cb.wait()
            jbase = jnp.full((_LANES,), j * _C, jnp.int32)

            def edge(i, c2):
                s = (a_v[i, pl.ds(0, _LANES)] * b_v[i, pl.ds(0, _LANES)])
                for c in range(1, cvecs):
                    s = s + (a_v[i, pl.ds(c * _LANES, _LANES)]
                             * b_v[i, pl.ds(c * _LANES, _LANES)])
                cs = plsc.cumsum(s)
                plsc.store_scatter(o_v, [jbase + i], cs, mask=last)
                return c2

            lax.fori_loop(0, _C, edge, 0)
            return carry

        lax.fori_loop(0, _NCH, chunk, 0)
        pltpu.sync_copy(o_v, out_hbm.at[pl.ds(wid * _EPW, _EPW)])

    return k(zn, srcp, dstp)


def kernel(x, edge_index, edge_weight, batch, emb_table, W1_rel, b1_rel,
           W1_root, Wmu_rel, bmu_rel, Wmu_root, Wstd_rel, bstd_rel, Wstd_root,
           Wc1, bc1, Wc2, bc2, log_std, eps):
    src = edge_index[0]
    dst = edge_index[1]

    n_pad = 10240
    x_pad = jnp.pad(x, (0, n_pad - N))
    e_raw = _sc_gather(emb_table, x_pad, n_pad, HIDDEN)[:N]
    e, eroot = _tc1(e_raw, W1_root, b1_rel)

    pe = _E_PAD - E
    srcp = jnp.pad(src, (0, pe)).reshape(_NW, _NCH, _C)
    dstp = jnp.pad(dst, (0, pe)).reshape(_NW, _NCH, _C)
    ewp = jnp.pad(edge_weight, (0, pe)).reshape(_NW, _EPW)
    zeros_tbl = jnp.zeros((_NROW, HIDDEN), jnp.float32)

    agg1p = _sc_segsum(e, srcp, dstp, ewp, zeros_tbl, HIDDEN)
    h = _tc3(agg1p, W1_rel, eroot)

    agg2p = _sc_segsum(h, srcp, dstp, ewp, zeros_tbl, EMB1)
    z, z_mu, z_std, zn, y, w_std = _tc5(
        agg2p, h, Wmu_rel, bmu_rel, Wmu_root, Wstd_rel, bstd_rel, Wstd_root,
        eps, batch, Wc1, bc1, Wc2, bc2, log_std)

    w_mu = _sc_edge_dot(zn, srcp, dstp)[:E]
    return (y, w_mu, w_std, z, z_mu, z_std)


# speedup vs baseline: 5.1192x; 1.3486x over previous
import functools

import jax
import jax.numpy as jnp
from jax import lax
from jax.experimental import pallas as pl
from jax.experimental.pallas import tpu as pltpu
from jax.experimental.pallas import tpu_sc as plsc

_NC = 2
_NS = 16
_NW = _NC * _NS
_LANES = 16

N = 10000
E = 320000
HIDDEN = 128
EMB1 = 128
EMB2 = 64
L1 = 64
G = 64


def _tc1_body(e_raw_ref, w1root_ref, b1_ref, e_ref, eroot_ref):
    e_raw = e_raw_ref[:]
    nrm2 = jnp.sum(e_raw * e_raw, axis=1, keepdims=True)
    scale = jnp.where(nrm2 > 1.0, lax.rsqrt(nrm2), 1.0)
    e = e_raw * scale
    e_ref[:] = e
    eroot_ref[:] = (
        jax.lax.dot_general(e, w1root_ref[:], (((1,), (1,)), ((), ())),
                            preferred_element_type=jnp.float32)
        + b1_ref[:][None, :]
    )


def _tc1(e_raw, W1_root, b1_rel):
    return pl.pallas_call(
        _tc1_body,
        out_shape=(
            jax.ShapeDtypeStruct((N, HIDDEN), jnp.float32),
            jax.ShapeDtypeStruct((N, EMB1), jnp.float32),
        ),
    )(e_raw, W1_root, b1_rel)


def _mm_halves(aggp_ref, w):
    def mm(a, wslice):
        return jax.lax.dot_general(a, wslice, (((1,), (1,)), ((), ())),
                                   preferred_element_type=jnp.float32)
    return (mm(aggp_ref[0:N, :], w[:, 0:_HC])
            + mm(aggp_ref[_NROW:_NROW + N, :], w[:, _HC:2 * _HC]))


def _tc3_body(aggp_ref, w1rel_ref, eroot_ref, h_ref):
    h = _mm_halves(aggp_ref, w1rel_ref[:]) + eroot_ref[:]
    h_ref[:] = jnp.maximum(h, 0.0)


def _tc3(agg1p, W1_rel, eroot):
    return pl.pallas_call(
        _tc3_body,
        out_shape=jax.ShapeDtypeStruct((N, EMB1), jnp.float32),
    )(agg1p, W1_rel, eroot)


def _split_cols(t):
    return jnp.stack([t[:, :_HC], t[:, _HC:]])


def _tc5_body(agg2_ref, h_ref, wmu_rel_ref, bmu_ref, wmu_root_ref,
              wstd_rel_ref, bstd_ref, wstd_root_ref, eps_ref, batch_ref,
              wc1_ref, bc1_ref, wc2_ref, bc2_ref, logstd_ref,
              z_ref, zmu_ref, zstd_ref, zn_ref, y_ref, wstd_out_ref):
    h = h_ref[:]

    def mm_t(a, w):
        return jax.lax.dot_general(a, w, (((1,), (1,)), ((), ())),
                                   preferred_element_type=jnp.float32)

    z_mu = jnp.tanh(_mm_halves(agg2_ref, wmu_rel_ref[:]) + bmu_ref[:][None, :]
                    + mm_t(h, wmu_root_ref[:]))
    z_ls = jnp.tanh(_mm_halves(agg2_ref, wstd_rel_ref[:]) + bstd_ref[:][None, :]
                    + mm_t(h, wstd_root_ref[:]))
    z_std = jnp.exp(z_ls)
    z = z_mu + z_std * eps_ref[:]
    zmu_ref[:] = z_mu
    zstd_ref[:] = z_std
    z_ref[:] = z
    zn2 = jnp.sum(z * z, axis=1, keepdims=True)
    rinv = 1.0 / jnp.maximum(jnp.sqrt(zn2), 1e-8)
    zn_ref[:] = z * rinv
    seg = lax.broadcasted_iota(jnp.int32, (G, N), 0)
    mask = (batch_ref[:][None, :] == seg).astype(jnp.float32)
    cnt = jnp.sum(mask, axis=1, keepdims=True)
    pooled = jax.lax.dot_general(mask, z_mu, (((1,), (0,)), ((), ())),
                                 preferred_element_type=jnp.float32)
    pooled = pooled / jnp.maximum(cnt, 1.0)
    y = jnp.maximum(mm_t(pooled, wc1_ref[:]) + bc1_ref[:][None, :], 0.0)
    y = mm_t(y, wc2_ref[:]) + bc2_ref[:][None, :]
    y = y - jnp.max(y, axis=1, keepdims=True)
    ey = jnp.exp(y)
    y_ref[:] = ey / jnp.sum(ey, axis=1, keepdims=True)
    wstd_out_ref[:] = jnp.exp(logstd_ref[:])


def _tc5(agg2, h, Wmu_rel, bmu_rel, Wmu_root, Wstd_rel, bstd_rel, Wstd_root,
         eps, batch, Wc1, bc1, Wc2, bc2, log_std):
    return pl.pallas_call(
        _tc5_body,
        out_shape=(
            jax.ShapeDtypeStruct((N, EMB2), jnp.float32),
            jax.ShapeDtypeStruct((N, EMB2), jnp.float32),
            jax.ShapeDtypeStruct((N, EMB2), jnp.float32),
            jax.ShapeDtypeStruct((N, EMB2), jnp.float32),
            jax.ShapeDtypeStruct((G, 2), jnp.float32),
            jax.ShapeDtypeStruct((1,), jnp.float32),
        ),
    )(agg2, h, Wmu_rel, bmu_rel, Wmu_root, Wstd_rel, bstd_rel, Wstd_root,
      eps, batch, Wc1, bc1, Wc2, bc2, log_std)


def _chunk_sizes(total, cap=128):
    out = []
    while total > 0:
        c = min(cap, total)
        out.append(c)
        total -= c
    return out


@functools.partial(jax.jit, static_argnames=("n_rows", "n_cols"))
def _sc_gather(table, idx, n_rows, n_cols):
    bpw = n_rows // _NW
    mesh = plsc.VectorSubcoreMesh(core_axis_name="c", subcore_axis_name="s")

    @functools.partial(
        pl.kernel, mesh=mesh,
        out_type=jax.ShapeDtypeStruct((n_rows, n_cols), jnp.float32),
        scratch_types=[
            pltpu.VMEM((bpw,), jnp.int32),
            pltpu.VMEM((bpw, n_cols), jnp.float32),
            pltpu.SemaphoreType.DMA,
        ],
    )
    def k(table_hbm, idx_hbm, out_hbm, idx_v, rows_v, sem):
        wid = lax.axis_index("s") * _NC + lax.axis_index("c")
        base = wid * bpw
        pltpu.sync_copy(idx_hbm.at[pl.ds(base, bpw)], idx_v)
        copies = []
        off = 0
        for cs in _chunk_sizes(bpw):
            copies.append(pltpu.async_copy(
                table_hbm.at[idx_v.at[pl.ds(off, cs)]],
                rows_v.at[pl.ds(off, cs)], sem))
            off += cs
        for c in copies:
            c.wait()
        pltpu.sync_copy(rows_v, out_hbm.at[pl.ds(base, bpw)])

    return k(table, idx)


def _gather_rows(table, idx):
    return table[idx]


_C = 128
_NCH = 79
_EPW = _NCH * _C
_E_PAD = _NW * _EPW
_NROW = 10240
_RPS = _NROW // _NS
_HC = 64
_NCH2 = _E_PAD // (_NS * _C)
_EPS = _NCH2 * _C


@jax.jit
def _sc_segsum(table_pair, srcp, dstp, ewp, zeros_half):
    mesh = plsc.VectorSubcoreMesh(core_axis_name="c", subcore_axis_name="s")
    cvecs = _HC // _LANES

    @functools.partial(
        pl.kernel, mesh=mesh,
        out_type=jax.ShapeDtypeStruct((2 * _NROW, _HC), jnp.float32),
        compiler_params=pltpu.CompilerParams(needs_layout_passes=False,
                                             use_tc_tiling_on_sc=False),
        scratch_types=[
            pltpu.VMEM((_NCH2, _C), jnp.int32),
            pltpu.VMEM((_NCH2, _C), jnp.int32),
            pltpu.VMEM((_EPS,), jnp.float32),
            pltpu.VMEM((_C, _HC), jnp.float32),
            pltpu.VMEM((_C, _HC), jnp.float32),
            pltpu.VMEM_SHARED((_NROW, _HC), jnp.float32),
            pltpu.SemaphoreType.DMA,
            pltpu.SemaphoreType.DMA,
        ],
    )
    def k(x_hbm, src_hbm, dst_hbm, ew_hbm, z_hbm, out_hbm,
          src_v, dst_v, ew_v, msg0_v, msg1_v, agg_sh, sem0, sem1):
        cid = lax.axis_index("c")
        sid = lax.axis_index("s")
        pltpu.sync_copy(z_hbm.at[pl.ds(sid * _RPS, _RPS)],
                        agg_sh.at[pl.ds(sid * _RPS, _RPS)])
        pltpu.sync_copy(src_hbm.at[sid], src_v)
        pltpu.sync_copy(dst_hbm.at[sid], dst_v)
        pltpu.sync_copy(ew_hbm.at[sid], ew_v)
        plsc.subcore_barrier()
        xh = x_hbm.at[cid]

        def scale_and_scatter(j, msg_v):
            jbase = jnp.full((_LANES,), j * _C, jnp.int32)

            def srow(i4, c2):
                for u in range(4):
                    i = i4 * 4 + u
                    w = plsc.load_gather(ew_v, [jbase + i])
                    for c in range(cvecs):
                        msg_v[i, pl.ds(c * _LANES, _LANES)] = (
                            msg_v[i, pl.ds(c * _LANES, _LANES)] * w)
                return c2

            lax.fori_loop(0, _C // 4, srow, 0)
            pltpu.sync_copy(msg_v, agg_sh.at[dst_v.at[j]], add=True)

        pltpu.async_copy(xh.at[src_v.at[0]], msg0_v, sem0)

        def pair(jj, carry):
            j = jj * 2
            c1 = pltpu.async_copy(xh.at[src_v.at[j + 1]], msg1_v, sem1)
            pltpu.make_async_copy(xh.at[src_v.at[j]], msg0_v, sem0).wait()
            scale_and_scatter(j, msg0_v)
            pltpu.async_copy(xh.at[src_v.at[j + 2]], msg0_v, sem0)
            c1.wait()
            scale_and_scatter(j + 1, msg1_v)
            return carry

        lax.fori_loop(0, _NCH2 // 2 - 1, pair, 0)
        jl = _NCH2 - 2
        cl = pltpu.async_copy(xh.at[src_v.at[jl + 1]], msg1_v, sem1)
        pltpu.make_async_copy(xh.at[src_v.at[jl]], msg0_v, sem0).wait()
        scale_and_scatter(jl, msg0_v)
        cl.wait()
        scale_and_scatter(jl + 1, msg1_v)

        plsc.subcore_barrier()
        pltpu.sync_copy(agg_sh.at[pl.ds(sid * _RPS, _RPS)],
                        out_hbm.at[pl.ds(cid * _NROW + sid * _RPS, _RPS)])

    return k(table_pair, srcp, dstp, ewp, zeros_half)


def _segsum(xrows, ew, dst):
    return jax.ops.segment_sum(xrows * ew[:, None], dst, num_segments=N)


@jax.jit
def _sc_edge_dot(zn, srcp, dstp):
    mesh = plsc.VectorSubcoreMesh(core_axis_name="c", subcore_axis_name="s")
    cvecs = EMB2 // _LANES

    @functools.partial(
        pl.kernel, mesh=mesh,
        out_type=jax.ShapeDtypeStruct((_E_PAD,), jnp.float32),
        compiler_params=pltpu.CompilerParams(needs_layout_passes=False,
                                             use_tc_tiling_on_sc=False),
        scratch_types=[
            pltpu.VMEM((_NCH, _C), jnp.int32),
            pltpu.VMEM((_NCH, _C), jnp.int32),
            pltpu.VMEM((_C, EMB2), jnp.float32),
            pltpu.VMEM((_C, EMB2), jnp.float32),
            pltpu.VMEM((_C, EMB2), jnp.float32),
            pltpu.VMEM((_C, EMB2), jnp.float32),
            pltpu.VMEM((_EPW,), jnp.float32),
            pltpu.SemaphoreType.DMA,
            pltpu.SemaphoreType.DMA,
        ],
    )
    def k(zn_hbm, src_hbm, dst_hbm, out_hbm, src_v, dst_v,
          a0_v, b0_v, a1_v, b1_v, o_v, sem0, sem1):
        cid = lax.axis_index("c")
        sid = lax.axis_index("s")
        wid = sid * _NC + cid
        pltpu.sync_copy(src_hbm.at[wid], src_v)
        pltpu.sync_copy(dst_hbm.at[wid], dst_v)
        lane = lax.broadcasted_iota(jnp.int32, (_LANES,), 0)
        last = lane == (_LANES - 1)

        def gather_ab(j, a_v, b_v, sem):
            pltpu.async_copy(zn_hbm.at[src_v.at[j]], a_v, sem)
            pltpu.async_copy(zn_hbm.at[dst_v.at[j]], b_v, sem)

        def wait_ab(j, a_v, b_v, sem):
            pltpu.make_async_copy(zn_hbm.at[src_v.at[j]], a_v, sem).wait()
            pltpu.make_async_copy(zn_hbm.at[dst_v.at[j]], b_v, sem).wait()

        def dots(j, a_v, b_v):
            jbase = jnp.full((_LANES,), j * _C, jnp.int32)

            def edge(i4, c2):
                for u in range(4):
                    i = i4 * 4 + u
                    s = (a_v[i, pl.ds(0, _LANES)] * b_v[i, pl.ds(0, _LANES)])
                    for c in range(1, cvecs):
                        s = s + (a_v[i, pl.ds(c * _LANES, _LANES)]
                                 * b_v[i, pl.ds(c * _LANES, _LANES)])
                    cs = plsc.cumsum(s)
                    plsc.store_scatter(o_v, [jbase + i], cs, mask=last)
                return c2

            lax.fori_loop(0, _C // 4, edge, 0)

        gather_ab(0, a0_v, b0_v, sem0)

        def pair(jj, carry):
            j = jj * 2
            gather_ab(j + 1, a1_v, b1_v, sem1)
            wait_ab(j, a0_v, b0_v, sem0)
            dots(j, a0_v, b0_v)
            gather_ab(j + 2, a0_v, b0_v, sem0)
            wait_ab(j + 1, a1_v, b1_v, sem1)
            dots(j + 1, a1_v, b1_v)
            return carry

        lax.fori_loop(0, (_NCH - 1) // 2, pair, 0)
        wait_ab(_NCH - 1, a0_v, b0_v, sem0)
        dots(_NCH - 1, a0_v, b0_v)
        pltpu.sync_copy(o_v, out_hbm.at[pl.ds(wid * _EPW, _EPW)])

    return k(zn, srcp, dstp)


def kernel(x, edge_index, edge_weight, batch, emb_table, W1_rel, b1_rel,
           W1_root, Wmu_rel, bmu_rel, Wmu_root, Wstd_rel, bstd_rel, Wstd_root,
           Wc1, bc1, Wc2, bc2, log_std, eps):
    src = edge_index[0]
    dst = edge_index[1]

    n_pad = 10240
    x_pad = jnp.pad(x, (0, n_pad - N))
    e_raw = _sc_gather(emb_table, x_pad, n_pad, HIDDEN)[:N]
    e, eroot = _tc1(e_raw, W1_root, b1_rel)

    pe = _E_PAD - E
    srcp = jnp.pad(src, (0, pe)).reshape(_NW, _NCH, _C)
    dstp = jnp.pad(dst, (0, pe)).reshape(_NW, _NCH, _C)
    srcp16 = srcp.reshape(_NS, _NCH2, _C)
    dstp16 = dstp.reshape(_NS, _NCH2, _C)
    ewp16 = jnp.pad(edge_weight, (0, pe)).reshape(_NS, _EPS)
    zeros_half = jnp.zeros((_NROW, _HC), jnp.float32)

    agg1p = _sc_segsum(_split_cols(e), srcp16, dstp16, ewp16, zeros_half)
    h = _tc3(agg1p, W1_rel, eroot)

    agg2p = _sc_segsum(_split_cols(h), srcp16, dstp16, ewp16, zeros_half)
    z, z_mu, z_std, zn, y, w_std = _tc5(
        agg2p, h, Wmu_rel, bmu_rel, Wmu_root, Wstd_rel, bstd_rel, Wstd_root,
        eps, batch, Wc1, bc1, Wc2, bc2, log_std)

    w_mu = _sc_edge_dot(zn, srcp, dstp)[:E]
    return (y, w_mu, w_std, z, z_mu, z_std)
